# baseline (speedup 1.0000x reference)
import jax
import jax.numpy as jnp
from jax import lax
from jax.experimental import pallas as pl

B = 128
D_IN = 256
N_LOC = 210

_GUMBEL = jax.random.gumbel(jax.random.key(42), (B, N_LOC), jnp.float32)


def _head_body(x_ref, wt_ref, b_ref, g_ref, probs_ref, loc_ref):
    logits = jnp.dot(x_ref[...], wt_ref[...],
                     preferred_element_type=jnp.float32) + b_ref[...]
    m = jnp.max(logits, axis=-1, keepdims=True)
    e = jnp.exp(logits - m)
    s = jnp.sum(e, axis=-1, keepdims=True)
    p = e / s
    probs_ref[...] = p
    scores = jnp.log(p + jnp.float32(1e-20)) + g_ref[...]
    best = jnp.max(scores, axis=-1, keepdims=True)
    idx = lax.broadcasted_iota(jnp.int32, (B, N_LOC), 1)
    cand = jnp.where(scores == best, idx, jnp.int32(N_LOC))
    loc_ref[...] = jnp.min(cand, axis=-1, keepdims=True)


def kernel(x, W, b, game_state, action_type):
    del game_state, action_type
    wt = W.T
    b2 = b.reshape(1, N_LOC)
    probs, loc2d = pl.pallas_call(
        _head_body,
        out_shape=(
            jax.ShapeDtypeStruct((B, N_LOC), jnp.float32),
            jax.ShapeDtypeStruct((B, 1), jnp.int32),
        ),
    )(x, wt, b2, _GUMBEL)
    return probs, loc2d[:, 0]


# speedup vs baseline: 1.3174x; 1.3174x over previous
import jax
import jax.numpy as jnp
import numpy as np
from jax import lax
from jax.experimental import pallas as pl

B = 128
D_IN = 256
N_LOC = 210


def _np_threefry2x32(k1, k2, x0, x1):
    def rotl(x, d):
        return ((x << np.uint32(d)) | (x >> np.uint32(32 - d))).astype(np.uint32)

    rot_a = (13, 15, 26, 6)
    rot_b = (17, 29, 16, 24)
    ks = (np.uint32(k1), np.uint32(k2),
          np.uint32(k1) ^ np.uint32(k2) ^ np.uint32(0x1BD11BDA))
    x0 = (x0 + ks[0]).astype(np.uint32)
    x1 = (x1 + ks[1]).astype(np.uint32)
    for j, rots in enumerate((rot_a, rot_b, rot_a, rot_b, rot_a)):
        for r in rots:
            x0 = (x0 + x1).astype(np.uint32)
            x1 = x0 ^ rotl(x1, r)
        x0 = (x0 + ks[(j + 1) % 3]).astype(np.uint32)
        x1 = (x1 + ks[(j + 2) % 3] + np.uint32(j + 1)).astype(np.uint32)
    return x0, x1


def _gumbel_const(seed, shape):
    n = int(np.prod(shape))
    counts_lo = np.arange(n, dtype=np.uint32).reshape(shape)
    counts_hi = np.zeros(shape, dtype=np.uint32)
    b0, b1 = _np_threefry2x32(0, seed, counts_hi, counts_lo)
    bits = b0 ^ b1
    float_bits = (bits >> np.uint32(9)) | np.uint32(0x3F800000)
    u01 = float_bits.view(np.float32) - np.float32(1.0)
    tiny = np.float32(np.finfo(np.float32).tiny)
    u = np.maximum(tiny, (u01 * (np.float32(1.0) - tiny) + tiny).astype(np.float32))
    return (-np.log(-np.log(u))).astype(np.float32)


_GUMBEL = _gumbel_const(42, (B, N_LOC))


def _head_body(x_ref, w_ref, b_ref, g_ref, probs_ref, loc_ref):
    logits = lax.dot_general(
        x_ref[...], w_ref[...],
        dimension_numbers=(((1,), (1,)), ((), ())),
        preferred_element_type=jnp.float32) + b_ref[...]
    m = jnp.max(logits, axis=-1, keepdims=True)
    e = jnp.exp(logits - m)
    s = jnp.sum(e, axis=-1, keepdims=True)
    p = e / s
    probs_ref[...] = p
    scores = jnp.log(p + jnp.float32(1e-20)) + g_ref[...]
    best = jnp.max(scores, axis=-1, keepdims=True)
    idx = lax.broadcasted_iota(jnp.int32, (B, N_LOC), 1)
    cand = jnp.where(scores == best, idx, jnp.int32(N_LOC))
    loc_ref[...] = jnp.min(cand, axis=-1, keepdims=True)


def kernel(x, W, b, game_state, action_type):
    del game_state, action_type
    b2 = b.reshape(1, N_LOC)
    probs, loc2d = pl.pallas_call(
        _head_body,
        out_shape=(
            jax.ShapeDtypeStruct((B, N_LOC), jnp.float32),
            jax.ShapeDtypeStruct((B, 1), jnp.int32),
        ),
    )(x, W, b2, jnp.asarray(_GUMBEL))
    return probs, loc2d[:, 0]


# speedup vs baseline: 1.5222x; 1.1555x over previous
import jax
import jax.numpy as jnp
import numpy as np
from jax import lax
from jax.experimental import pallas as pl

B = 128
D_IN = 256
N_LOC = 210


def _np_threefry2x32(k1, k2, x0, x1):
    def rotl(x, d):
        return ((x << np.uint32(d)) | (x >> np.uint32(32 - d))).astype(np.uint32)

    rot_a = (13, 15, 26, 6)
    rot_b = (17, 29, 16, 24)
    ks = (np.uint32(k1), np.uint32(k2),
          np.uint32(k1) ^ np.uint32(k2) ^ np.uint32(0x1BD11BDA))
    x0 = (x0 + ks[0]).astype(np.uint32)
    x1 = (x1 + ks[1]).astype(np.uint32)
    for j, rots in enumerate((rot_a, rot_b, rot_a, rot_b, rot_a)):
        for r in rots:
            x0 = (x0 + x1).astype(np.uint32)
            x1 = x0 ^ rotl(x1, r)
        x0 = (x0 + ks[(j + 1) % 3]).astype(np.uint32)
        x1 = (x1 + ks[(j + 2) % 3] + np.uint32(j + 1)).astype(np.uint32)
    return x0, x1


def _gumbel_const(seed, shape):
    n = int(np.prod(shape))
    counts_lo = np.arange(n, dtype=np.uint32).reshape(shape)
    counts_hi = np.zeros(shape, dtype=np.uint32)
    b0, b1 = _np_threefry2x32(0, seed, counts_hi, counts_lo)
    bits = b0 ^ b1
    float_bits = (bits >> np.uint32(9)) | np.uint32(0x3F800000)
    u01 = float_bits.view(np.float32) - np.float32(1.0)
    tiny = np.float32(np.finfo(np.float32).tiny)
    u = np.maximum(tiny, (u01 * (np.float32(1.0) - tiny) + tiny).astype(np.float32))
    return (-np.log(-np.log(u))).astype(np.float32)


_GUMBEL = _gumbel_const(42, (B, N_LOC))


def _head_body(x_ref, w_ref, b_ref, g_ref, probs_ref, loc_ref):
    logits = lax.dot_general(
        x_ref[...], w_ref[...],
        dimension_numbers=(((1,), (1,)), ((), ())),
        preferred_element_type=jnp.float32) + b_ref[...]
    m = jnp.max(logits, axis=-1, keepdims=True)
    e = jnp.exp(logits - m)
    s = jnp.sum(e, axis=-1, keepdims=True)
    p = e / s
    probs_ref[...] = p
    scores = jnp.log(p + jnp.float32(1e-20)) + g_ref[...]
    best = jnp.max(scores, axis=-1, keepdims=True)
    idx = lax.broadcasted_iota(jnp.int32, (B, N_LOC), 1)
    cand = jnp.where(scores == best, idx, jnp.int32(N_LOC))
    loc_ref[...] = jnp.min(cand, axis=-1)


def kernel(x, W, b, game_state, action_type):
    del game_state, action_type
    b2 = b.reshape(1, N_LOC)
    probs, loc = pl.pallas_call(
        _head_body,
        out_shape=(
            jax.ShapeDtypeStruct((B, N_LOC), jnp.float32),
            jax.ShapeDtypeStruct((B,), jnp.int32),
        ),
    )(x, W, b2, jnp.asarray(_GUMBEL))
    return probs, loc
